# baseline (speedup 1.0000x reference)
import jax
import jax.numpy as jnp
from jax import lax
from jax.experimental import pallas as pl
from jax.experimental.pallas import tpu as pltpu
from jax.experimental.pallas import tpu_sc as plsc

_NC, _NS = 2, 16
_NW = _NC * _NS
_B, _K, _N = 2, 256, 17
_BN = _B * _N
_H = 64
_ROW = 128
_CH = _H * _ROW
_CPC = 32
_BIG = 1e30


def _all_nonzero(vec):
    bits = [vec[j] != 0.0 for j in range(16)]
    while len(bits) > 1:
        bits = [jnp.logical_and(bits[i], bits[i + 1]) for i in range(0, len(bits), 2)]
    return bits[0]


def _round_clamp(v, lo, hi):
    t = jnp.minimum(jnp.maximum(v, lo), hi)
    return (t + 0.5).astype(jnp.int32).astype(jnp.float32)


def _prep_body(hm_ref, kp_ref, scores_ref, hmp_ref, flg_ref, kp8_ref, st_ref):
    hm = hm_ref[...]
    hmp_ref[...] = jnp.pad(hm, ((0, 0), (0, 0), (0, 0), (0, _ROW - _H)))
    gmin = jnp.min(jnp.abs(hm), axis=(2, 3)).reshape(_BN)
    flg = jnp.where(gmin == 0.0, 1.0, 0.0)
    flg_ref[...] = jnp.pad(flg, (0, 64 - _BN)).reshape(1, 64)
    kpt = jnp.transpose(kp_ref[...], (0, 2, 1)).reshape(4, _K)
    kp8_ref[...] = jnp.pad(kpt, ((0, 4), (0, 0)))
    st_ref[...] = jnp.transpose(scores_ref[...], (0, 2, 1)).reshape(_BN, _K)


def _do_channel(ch, flag_sparse, hm_hbm, hm_v, kp_v, res_v, out_hbm):
    b = ch // _N
    kbase = b * (2 * _K)

    def load_kp(kc):
        ky = kp_v[pl.ds(kbase + kc * 16, 16)]
        kx = kp_v[pl.ds(kbase + _K + kc * 16, 16)]
        return ky, kx

    @pl.when(jnp.logical_not(flag_sparse))
    def _():
        for kc in range(_K // 16):
            ky, kx = load_kp(kc)
            dy = ky - _round_clamp(ky, 0.0, 63.0)
            dx = kx - _round_clamp(kx, 0.0, 63.0)
            res_v[pl.ds(kc * 16, 16)] = dy * dy + dx * dx

    @pl.when(flag_sparse)
    def _():
        pltpu.sync_copy(hm_hbm.at[pl.ds(ch * _CH, _CH)], hm_v)
        for kc in range(_K // 16):
            res_v[pl.ds(kc * 16, 16)] = jnp.full((16,), _BIG, jnp.float32)

        def chunk_body(c, carry):
            base = c * (2 * _ROW)
            y0f = (2 * c).astype(jnp.float32)
            cabs = jnp.abs(hm_v[pl.ds(base, 16)])
            for j in range(1, 8):
                off = base + (j // 4) * _ROW + (j % 4) * 16
                cabs = jnp.minimum(cabs, jnp.abs(hm_v[pl.ds(off, 16)]))
            dense_c = _all_nonzero(cabs)

            @pl.when(dense_c)
            def _():
                def fast_kc(kc, cc):
                    ky, kx = load_kp(kc)
                    ty = jnp.minimum(jnp.maximum(ky, y0f), y0f + 1.0)
                    ys = jnp.where(ty >= y0f + 0.5, y0f + 1.0, y0f)
                    dy = ky - ys
                    dx = kx - _round_clamp(kx, 0.0, 63.0)
                    d2 = dy * dy + dx * dx
                    res_v[pl.ds(kc * 16, 16)] = jnp.minimum(
                        res_v[pl.ds(kc * 16, 16)], d2)
                    return cc

                lax.fori_loop(0, _K // 16, fast_kc, 0)

            @pl.when(jnp.logical_not(dense_c))
            def _():
                def slow_kc(kc, cc):
                    ky, kx = load_kp(kc)

                    def grp(gi, acc):
                        off = base + (gi // 4) * _ROW + (gi % 4) * 16
                        hv = hm_v[pl.ds(off, 16)]
                        pen_v = jnp.where(hv != 0.0, 0.0, _BIG)
                        ypg = y0f + (gi // 4).astype(jnp.float32)
                        xpg = ((gi % 4) * 16).astype(jnp.float32)
                        for lane in range(16):
                            dy = ky - ypg
                            dx = kx - (xpg + float(lane))
                            acc = jnp.minimum(acc, dy * dy + dx * dx + pen_v[lane])
                        return acc

                    acc = lax.fori_loop(0, 8, grp, jnp.full((16,), _BIG, jnp.float32))
                    res_v[pl.ds(kc * 16, 16)] = jnp.minimum(
                        res_v[pl.ds(kc * 16, 16)], acc)
                    return cc

                lax.fori_loop(0, _K // 16, slow_kc, 0)

            return carry

        lax.fori_loop(0, _CPC, chunk_body, 0)

    pltpu.sync_copy(res_v, out_hbm.at[ch])


def _sc_body(hm_hbm, kp_hbm, flg_hbm, out_hbm, hm_v, kp_v, flg_v, res_v, sem_kp, sem_fl):
    wid = lax.axis_index("s") * _NC + lax.axis_index("c")
    second = wid < _BN - _NW
    h_kp = pltpu.async_copy(kp_hbm.at[pl.ds(0, 2 * _B * _K)], kp_v, sem_kp)
    h_fl = pltpu.async_copy(flg_hbm, flg_v, sem_fl)
    h_fl.wait()
    h_kp.wait()
    flag1 = flg_v[pl.ds(wid, 16)][0] != 0.0
    _do_channel(wid, flag1, hm_hbm, hm_v, kp_v, res_v, out_hbm)

    @pl.when(second)
    def _():
        flag2 = flg_v[pl.ds(_NW + wid, 16)][0] != 0.0
        _do_channel(_NW + wid, flag2, hm_hbm, hm_v, kp_v, res_v, out_hbm)


def _loss_body(d2_ref, scores_ref, out_ref):
    d = jnp.sqrt(d2_ref[...])
    s = scores_ref[...]
    pos = d < 1.0
    safe_d = jnp.where(pos, d, 0.0)
    safe_s = jnp.where(pos, s, 1.0)
    pos_loss = jnp.sum(
        jnp.where(pos, 10000.0 / (1.0 + jnp.exp(safe_d)) * jnp.log(safe_s), 0.0))
    safe_ns = jnp.where(pos, 0.5, 1.0 - s)
    neg_loss = jnp.sum(jnp.where(pos, 0.0, jnp.log(safe_ns)))
    neg_count = jnp.sum(jnp.logical_not(pos).astype(jnp.float32))
    loss = -pos_loss
    loss = jnp.where(neg_count > 0, loss - 10000.0 / neg_count * neg_loss, loss)
    out_ref[0, 0] = loss


def kernel(all_scores, gt_heatmap, keypoints_list):
    hm_pad, flg, kp8, scores_t = pl.pallas_call(
        _prep_body,
        out_shape=[
            jax.ShapeDtypeStruct((_B, _N, _H, _ROW), jnp.float32),
            jax.ShapeDtypeStruct((1, 64), jnp.float32),
            jax.ShapeDtypeStruct((8, _K), jnp.float32),
            jax.ShapeDtypeStruct((_BN, _K), jnp.float32),
        ],
        in_specs=[
            pl.BlockSpec(memory_space=pltpu.VMEM),
            pl.BlockSpec(memory_space=pltpu.VMEM),
            pl.BlockSpec(memory_space=pltpu.VMEM),
        ],
        out_specs=[
            pl.BlockSpec(memory_space=pltpu.VMEM),
            pl.BlockSpec(memory_space=pltpu.VMEM),
            pl.BlockSpec(memory_space=pltpu.VMEM),
            pl.BlockSpec(memory_space=pltpu.VMEM),
        ],
    )(gt_heatmap, keypoints_list, all_scores)
    mesh = plsc.VectorSubcoreMesh(
        core_axis_name="c", subcore_axis_name="s", num_cores=_NC, num_subcores=_NS)
    d2 = pl.kernel(
        _sc_body,
        out_type=jax.ShapeDtypeStruct((_BN, _K), jnp.float32),
        mesh=mesh,
        scratch_types=[
            pltpu.VMEM((_CH,), jnp.float32),
            pltpu.VMEM((2 * _B * _K,), jnp.float32),
            pltpu.VMEM((64,), jnp.float32),
            pltpu.VMEM((_K,), jnp.float32),
            pltpu.SemaphoreType.DMA,
            pltpu.SemaphoreType.DMA,
        ],
    )(hm_pad.reshape(_BN * _CH), kp8.reshape(8 * _K), flg.reshape(64))
    out = pl.pallas_call(
        _loss_body,
        out_shape=jax.ShapeDtypeStruct((1, 1), jnp.float32),
        in_specs=[
            pl.BlockSpec(memory_space=pltpu.VMEM),
            pl.BlockSpec(memory_space=pltpu.VMEM),
        ],
        out_specs=pl.BlockSpec(memory_space=pltpu.SMEM),
    )(d2, scores_t)
    return out[0, 0]


# speedup vs baseline: 1.0525x; 1.0525x over previous
import jax
import jax.numpy as jnp
from jax import lax
from jax.experimental import pallas as pl
from jax.experimental.pallas import tpu as pltpu
from jax.experimental.pallas import tpu_sc as plsc

_NC, _NS = 2, 16
_NW = _NC * _NS
_B, _K, _N = 2, 256, 17
_BN = _B * _N
_H = 64
_ROW = 128
_CH = _H * _ROW
_CPC = 32
_BIG = 1e30


def _all_nonzero(vec):
    bits = [vec[j] != 0.0 for j in range(16)]
    while len(bits) > 1:
        bits = [jnp.logical_and(bits[i], bits[i + 1]) for i in range(0, len(bits), 2)]
    return bits[0]


def _round_clamp(v, lo, hi):
    t = jnp.minimum(jnp.maximum(v, lo), hi)
    return (t + 0.5).astype(jnp.int32).astype(jnp.float32)


def _do_channel(ch, hm_v, kp_v, res_v, out_hbm):
    b = ch // _N
    kbase = b * (2 * _K)

    def load_kp(kc):
        ky = kp_v[pl.ds(kbase + kc * 16, 16)]
        kx = kp_v[pl.ds(kbase + _K + kc * 16, 16)]
        return ky, kx

    def dens(i, m):
        for j in range(16):
            off = (i * 4 + j // 4) * _ROW + (j % 4) * 16
            m = jnp.minimum(m, jnp.abs(hm_v[pl.ds(off, 16)]))
        return m

    mabs = lax.fori_loop(0, 16, dens, jnp.full((16,), _BIG, jnp.float32))
    dense_all = _all_nonzero(mabs)

    @pl.when(dense_all)
    def _():
        for kc in range(_K // 16):
            ky, kx = load_kp(kc)
            dy = ky - _round_clamp(ky, 0.0, 63.0)
            dx = kx - _round_clamp(kx, 0.0, 63.0)
            res_v[pl.ds(kc * 16, 16)] = dy * dy + dx * dx

    @pl.when(jnp.logical_not(dense_all))
    def _():
        for kc in range(_K // 16):
            res_v[pl.ds(kc * 16, 16)] = jnp.full((16,), _BIG, jnp.float32)

        def chunk_body(c, carry):
            base = c * (2 * _ROW)
            y0f = (2 * c).astype(jnp.float32)
            cabs = jnp.abs(hm_v[pl.ds(base, 16)])
            for j in range(1, 8):
                off = base + (j // 4) * _ROW + (j % 4) * 16
                cabs = jnp.minimum(cabs, jnp.abs(hm_v[pl.ds(off, 16)]))
            dense_c = _all_nonzero(cabs)

            @pl.when(dense_c)
            def _():
                def fast_kc(kc, cc):
                    ky, kx = load_kp(kc)
                    ty = jnp.minimum(jnp.maximum(ky, y0f), y0f + 1.0)
                    ys = jnp.where(ty >= y0f + 0.5, y0f + 1.0, y0f)
                    dy = ky - ys
                    dx = kx - _round_clamp(kx, 0.0, 63.0)
                    d2 = dy * dy + dx * dx
                    res_v[pl.ds(kc * 16, 16)] = jnp.minimum(
                        res_v[pl.ds(kc * 16, 16)], d2)
                    return cc

                lax.fori_loop(0, _K // 16, fast_kc, 0)

            @pl.when(jnp.logical_not(dense_c))
            def _():
                def slow_kc(kc, cc):
                    ky, kx = load_kp(kc)

                    def grp(gi, acc):
                        off = base + (gi // 4) * _ROW + (gi % 4) * 16
                        hv = hm_v[pl.ds(off, 16)]
                        pen_v = jnp.where(hv != 0.0, 0.0, _BIG)
                        ypg = y0f + (gi // 4).astype(jnp.float32)
                        xpg = ((gi % 4) * 16).astype(jnp.float32)
                        for lane in range(16):
                            dy = ky - ypg
                            dx = kx - (xpg + float(lane))
                            acc = jnp.minimum(acc, dy * dy + dx * dx + pen_v[lane])
                        return acc

                    acc = lax.fori_loop(0, 8, grp, jnp.full((16,), _BIG, jnp.float32))
                    res_v[pl.ds(kc * 16, 16)] = jnp.minimum(
                        res_v[pl.ds(kc * 16, 16)], acc)
                    return cc

                lax.fori_loop(0, _K // 16, slow_kc, 0)

            return carry

        lax.fori_loop(0, _CPC, chunk_body, 0)

    pltpu.sync_copy(res_v, out_hbm.at[ch])


def _sc_body(hm_hbm, kp_hbm, out_hbm, hm_v, hm_v2, kp_v, res_v, sem_kp, sem1, sem2):
    wid = lax.axis_index("s") * _NC + lax.axis_index("c")
    second = wid < _BN - _NW
    h_kp = pltpu.async_copy(kp_hbm, kp_v, sem_kp)
    h1 = pltpu.async_copy(hm_hbm.at[pl.ds(wid * _CH, _CH)], hm_v, sem1)

    @pl.when(second)
    def _():
        pltpu.async_copy(hm_hbm.at[pl.ds((_NW + wid) * _CH, _CH)], hm_v2, sem2)

    h_kp.wait()
    h1.wait()
    _do_channel(wid, hm_v, kp_v, res_v, out_hbm)

    @pl.when(second)
    def _():
        pltpu.make_async_copy(
            hm_hbm.at[pl.ds((_NW + wid) * _CH, _CH)], hm_v2, sem2).wait()
        _do_channel(_NW + wid, hm_v2, kp_v, res_v, out_hbm)


def _sc_dense_body(kp_hbm, out_hbm, kp_v, res_v):
    wid = lax.axis_index("s") * _NC + lax.axis_index("c")
    pltpu.sync_copy(kp_hbm, kp_v)

    def closed(ch):
        kbase = (ch // _N) * (2 * _K)
        for kc in range(_K // 16):
            ky = kp_v[pl.ds(kbase + kc * 16, 16)]
            kx = kp_v[pl.ds(kbase + _K + kc * 16, 16)]
            dy = ky - _round_clamp(ky, 0.0, 63.0)
            dx = kx - _round_clamp(kx, 0.0, 63.0)
            res_v[pl.ds(kc * 16, 16)] = dy * dy + dx * dx
        pltpu.sync_copy(res_v, out_hbm.at[ch])

    closed(wid)

    @pl.when(wid < _BN - _NW)
    def _():
        closed(_NW + wid)


def _loss_body(d2_ref, scores_ref, out_ref):
    d = jnp.sqrt(d2_ref[...])
    s = scores_ref[...]
    pos = d < 1.0
    safe_d = jnp.where(pos, d, 0.0)
    safe_s = jnp.where(pos, s, 1.0)
    pos_loss = jnp.sum(
        jnp.where(pos, 10000.0 / (1.0 + jnp.exp(safe_d)) * jnp.log(safe_s), 0.0))
    safe_ns = jnp.where(pos, 0.5, 1.0 - s)
    neg_loss = jnp.sum(jnp.where(pos, 0.0, jnp.log(safe_ns)))
    neg_count = jnp.sum(jnp.logical_not(pos).astype(jnp.float32))
    loss = -pos_loss
    loss = jnp.where(neg_count > 0, loss - 10000.0 / neg_count * neg_loss, loss)
    out_ref[0, 0] = loss


def kernel(all_scores, gt_heatmap, keypoints_list):
    kp_flat = keypoints_list.transpose(0, 2, 1).reshape(_B * 2 * _K)
    scores_t = all_scores.transpose(0, 2, 1).reshape(_BN, _K)
    all_dense = jnp.all(gt_heatmap != 0.0)
    mesh = plsc.VectorSubcoreMesh(
        core_axis_name="c", subcore_axis_name="s", num_cores=_NC, num_subcores=_NS)

    def _dense_branch(kp, hm):
        return pl.kernel(
            _sc_dense_body,
            out_type=jax.ShapeDtypeStruct((_BN, _K), jnp.float32),
            mesh=mesh,
            scratch_types=[
                pltpu.VMEM((_B * 2 * _K,), jnp.float32),
                pltpu.VMEM((_K,), jnp.float32),
            ],
        )(kp)

    def _sparse_branch(kp, hm):
        hm_pad = jnp.pad(hm, ((0, 0), (0, 0), (0, 0), (0, _ROW - _H)))
        hm_flat = hm_pad.reshape(_BN * _CH)
        return pl.kernel(
            _sc_body,
            out_type=jax.ShapeDtypeStruct((_BN, _K), jnp.float32),
            mesh=mesh,
            scratch_types=[
                pltpu.VMEM((_CH,), jnp.float32),
                pltpu.VMEM((_CH,), jnp.float32),
                pltpu.VMEM((_B * 2 * _K,), jnp.float32),
                pltpu.VMEM((_K,), jnp.float32),
                pltpu.SemaphoreType.DMA,
                pltpu.SemaphoreType.DMA,
                pltpu.SemaphoreType.DMA,
            ],
        )(hm_flat, kp)

    d2 = lax.cond(all_dense, _dense_branch, _sparse_branch, kp_flat, gt_heatmap)
    out = pl.pallas_call(
        _loss_body,
        out_shape=jax.ShapeDtypeStruct((1, 1), jnp.float32),
        in_specs=[
            pl.BlockSpec(memory_space=pltpu.VMEM),
            pl.BlockSpec(memory_space=pltpu.VMEM),
        ],
        out_specs=pl.BlockSpec(memory_space=pltpu.SMEM),
    )(d2, scores_t)
    return out[0, 0]


# speedup vs baseline: 1.0649x; 1.0118x over previous
import jax
import jax.numpy as jnp
from jax import lax
from jax.experimental import pallas as pl
from jax.experimental.pallas import tpu as pltpu
from jax.experimental.pallas import tpu_sc as plsc

_NC, _NS = 2, 16
_NW = _NC * _NS
_B, _K, _N = 2, 256, 17
_BN = _B * _N
_H = 64
_ROW = 128
_CH = _H * _ROW
_CPC = 32
_BIG = 1e30


def _all_nonzero(vec):
    bits = [vec[j] != 0.0 for j in range(16)]
    while len(bits) > 1:
        bits = [jnp.logical_and(bits[i], bits[i + 1]) for i in range(0, len(bits), 2)]
    return bits[0]


def _round_clamp(v, lo, hi):
    t = jnp.minimum(jnp.maximum(v, lo), hi)
    return (t + 0.5).astype(jnp.int32).astype(jnp.float32)


def _do_channel(ch, hm_v, kp_v, res_v, out_hbm):
    b = ch // _N
    kbase = b * (2 * _K)

    def load_kp(kc):
        ky = kp_v[pl.ds(kbase + kc * 16, 16)]
        kx = kp_v[pl.ds(kbase + _K + kc * 16, 16)]
        return ky, kx

    def dens(i, m):
        for j in range(16):
            off = (i * 4 + j // 4) * _ROW + (j % 4) * 16
            m = jnp.minimum(m, jnp.abs(hm_v[pl.ds(off, 16)]))
        return m

    mabs = lax.fori_loop(0, 16, dens, jnp.full((16,), _BIG, jnp.float32))
    dense_all = _all_nonzero(mabs)

    @pl.when(dense_all)
    def _():
        for kc in range(_K // 16):
            ky, kx = load_kp(kc)
            dy = ky - _round_clamp(ky, 0.0, 63.0)
            dx = kx - _round_clamp(kx, 0.0, 63.0)
            res_v[pl.ds(kc * 16, 16)] = dy * dy + dx * dx

    @pl.when(jnp.logical_not(dense_all))
    def _():
        for kc in range(_K // 16):
            res_v[pl.ds(kc * 16, 16)] = jnp.full((16,), _BIG, jnp.float32)

        def chunk_body(c, carry):
            base = c * (2 * _ROW)
            y0f = (2 * c).astype(jnp.float32)
            cabs = jnp.abs(hm_v[pl.ds(base, 16)])
            for j in range(1, 8):
                off = base + (j // 4) * _ROW + (j % 4) * 16
                cabs = jnp.minimum(cabs, jnp.abs(hm_v[pl.ds(off, 16)]))
            dense_c = _all_nonzero(cabs)

            @pl.when(dense_c)
            def _():
                def fast_kc(kc, cc):
                    ky, kx = load_kp(kc)
                    ty = jnp.minimum(jnp.maximum(ky, y0f), y0f + 1.0)
                    ys = jnp.where(ty >= y0f + 0.5, y0f + 1.0, y0f)
                    dy = ky - ys
                    dx = kx - _round_clamp(kx, 0.0, 63.0)
                    d2 = dy * dy + dx * dx
                    res_v[pl.ds(kc * 16, 16)] = jnp.minimum(
                        res_v[pl.ds(kc * 16, 16)], d2)
                    return cc

                lax.fori_loop(0, _K // 16, fast_kc, 0)

            @pl.when(jnp.logical_not(dense_c))
            def _():
                def slow_kc(kc, cc):
                    ky, kx = load_kp(kc)

                    def grp(gi, acc):
                        off = base + (gi // 4) * _ROW + (gi % 4) * 16
                        hv = hm_v[pl.ds(off, 16)]
                        pen_v = jnp.where(hv != 0.0, 0.0, _BIG)
                        ypg = y0f + (gi // 4).astype(jnp.float32)
                        xpg = ((gi % 4) * 16).astype(jnp.float32)
                        for lane in range(16):
                            dy = ky - ypg
                            dx = kx - (xpg + float(lane))
                            acc = jnp.minimum(acc, dy * dy + dx * dx + pen_v[lane])
                        return acc

                    acc = lax.fori_loop(0, 8, grp, jnp.full((16,), _BIG, jnp.float32))
                    res_v[pl.ds(kc * 16, 16)] = jnp.minimum(
                        res_v[pl.ds(kc * 16, 16)], acc)
                    return cc

                lax.fori_loop(0, _K // 16, slow_kc, 0)

            return carry

        lax.fori_loop(0, _CPC, chunk_body, 0)

    pltpu.sync_copy(res_v, out_hbm.at[ch])


def _sc_body(hm_hbm, kp_hbm, out_hbm, hm_v, hm_v2, kp_v, res_v, sem_kp, sem1, sem2):
    wid = lax.axis_index("s") * _NC + lax.axis_index("c")
    second = wid < _BN - _NW
    h_kp = pltpu.async_copy(kp_hbm, kp_v, sem_kp)
    h1 = pltpu.async_copy(hm_hbm.at[pl.ds(wid * _CH, _CH)], hm_v, sem1)

    @pl.when(second)
    def _():
        pltpu.async_copy(hm_hbm.at[pl.ds((_NW + wid) * _CH, _CH)], hm_v2, sem2)

    h_kp.wait()
    h1.wait()
    _do_channel(wid, hm_v, kp_v, res_v, out_hbm)

    @pl.when(second)
    def _():
        pltpu.make_async_copy(
            hm_hbm.at[pl.ds((_NW + wid) * _CH, _CH)], hm_v2, sem2).wait()
        _do_channel(_NW + wid, hm_v2, kp_v, res_v, out_hbm)


def _dense_flag_body(hm_ref, out_ref):
    out_ref[0, 0] = jnp.min(jnp.abs(hm_ref[...]))


def _sc_dense_body(kp_hbm, out_hbm, kp_v, res_v):
    wid = lax.axis_index("s") * _NC + lax.axis_index("c")
    pltpu.sync_copy(kp_hbm, kp_v)

    def closed(ch):
        kbase = (ch // _N) * (2 * _K)
        for kc in range(_K // 16):
            ky = kp_v[pl.ds(kbase + kc * 16, 16)]
            kx = kp_v[pl.ds(kbase + _K + kc * 16, 16)]
            dy = ky - _round_clamp(ky, 0.0, 63.0)
            dx = kx - _round_clamp(kx, 0.0, 63.0)
            res_v[pl.ds(kc * 16, 16)] = dy * dy + dx * dx
        pltpu.sync_copy(res_v, out_hbm.at[ch])

    closed(wid)

    @pl.when(wid < _BN - _NW)
    def _():
        closed(_NW + wid)


def _loss_body(d2_ref, scores_ref, out_ref):
    d = jnp.sqrt(d2_ref[...])
    s = scores_ref[...]
    pos = d < 1.0
    safe_d = jnp.where(pos, d, 0.0)
    safe_s = jnp.where(pos, s, 1.0)
    pos_loss = jnp.sum(
        jnp.where(pos, 10000.0 / (1.0 + jnp.exp(safe_d)) * jnp.log(safe_s), 0.0))
    safe_ns = jnp.where(pos, 0.5, 1.0 - s)
    neg_loss = jnp.sum(jnp.where(pos, 0.0, jnp.log(safe_ns)))
    neg_count = jnp.sum(jnp.logical_not(pos).astype(jnp.float32))
    loss = -pos_loss
    loss = jnp.where(neg_count > 0, loss - 10000.0 / neg_count * neg_loss, loss)
    out_ref[0, 0] = loss


def kernel(all_scores, gt_heatmap, keypoints_list):
    kp_flat = keypoints_list.transpose(0, 2, 1).reshape(_B * 2 * _K)
    scores_t = all_scores.transpose(0, 2, 1).reshape(_BN, _K)
    min_abs = pl.pallas_call(
        _dense_flag_body,
        out_shape=jax.ShapeDtypeStruct((1, 1), jnp.float32),
        in_specs=[pl.BlockSpec(memory_space=pltpu.VMEM)],
        out_specs=pl.BlockSpec(memory_space=pltpu.SMEM),
    )(gt_heatmap)
    all_dense = min_abs[0, 0] != 0.0
    mesh = plsc.VectorSubcoreMesh(
        core_axis_name="c", subcore_axis_name="s", num_cores=_NC, num_subcores=_NS)

    def _dense_branch(kp, hm):
        return pl.kernel(
            _sc_dense_body,
            out_type=jax.ShapeDtypeStruct((_BN, _K), jnp.float32),
            mesh=mesh,
            scratch_types=[
                pltpu.VMEM((_B * 2 * _K,), jnp.float32),
                pltpu.VMEM((_K,), jnp.float32),
            ],
        )(kp)

    def _sparse_branch(kp, hm):
        hm_pad = jnp.pad(hm, ((0, 0), (0, 0), (0, 0), (0, _ROW - _H)))
        hm_flat = hm_pad.reshape(_BN * _CH)
        return pl.kernel(
            _sc_body,
            out_type=jax.ShapeDtypeStruct((_BN, _K), jnp.float32),
            mesh=mesh,
            scratch_types=[
                pltpu.VMEM((_CH,), jnp.float32),
                pltpu.VMEM((_CH,), jnp.float32),
                pltpu.VMEM((_B * 2 * _K,), jnp.float32),
                pltpu.VMEM((_K,), jnp.float32),
                pltpu.SemaphoreType.DMA,
                pltpu.SemaphoreType.DMA,
                pltpu.SemaphoreType.DMA,
            ],
        )(hm_flat, kp)

    d2 = lax.cond(all_dense, _dense_branch, _sparse_branch, kp_flat, gt_heatmap)
    out = pl.pallas_call(
        _loss_body,
        out_shape=jax.ShapeDtypeStruct((1, 1), jnp.float32),
        in_specs=[
            pl.BlockSpec(memory_space=pltpu.VMEM),
            pl.BlockSpec(memory_space=pltpu.VMEM),
        ],
        out_specs=pl.BlockSpec(memory_space=pltpu.SMEM),
    )(d2, scores_t)
    return out[0, 0]


# speedup vs baseline: 1.0850x; 1.0189x over previous
import jax
import jax.numpy as jnp
from jax import lax
from jax.experimental import pallas as pl
from jax.experimental.pallas import tpu as pltpu
from jax.experimental.pallas import tpu_sc as plsc

_NC, _NS = 2, 16
_NW = _NC * _NS
_B, _K, _N = 2, 256, 17
_BN = _B * _N
_H = 64
_ROW = 128
_CH = _H * _ROW
_CPC = 32
_BIG = 1e30


def _all_nonzero(vec):
    bits = [vec[j] != 0.0 for j in range(16)]
    while len(bits) > 1:
        bits = [jnp.logical_and(bits[i], bits[i + 1]) for i in range(0, len(bits), 2)]
    return bits[0]


def _round_clamp(v, lo, hi):
    t = jnp.minimum(jnp.maximum(v, lo), hi)
    return (t + 0.5).astype(jnp.int32).astype(jnp.float32)


def _do_channel(ch, hm_v, kp_v, res_v, out_hbm):
    b = ch // _N
    kbase = b * (2 * _K)

    def load_kp(kc):
        ky = kp_v[pl.ds(kbase + kc * 16, 16)]
        kx = kp_v[pl.ds(kbase + _K + kc * 16, 16)]
        return ky, kx

    def dens(i, m):
        for j in range(16):
            off = (i * 4 + j // 4) * _ROW + (j % 4) * 16
            m = jnp.minimum(m, jnp.abs(hm_v[pl.ds(off, 16)]))
        return m

    mabs = lax.fori_loop(0, 16, dens, jnp.full((16,), _BIG, jnp.float32))
    dense_all = _all_nonzero(mabs)

    @pl.when(dense_all)
    def _():
        for kc in range(_K // 16):
            ky, kx = load_kp(kc)
            dy = ky - _round_clamp(ky, 0.0, 63.0)
            dx = kx - _round_clamp(kx, 0.0, 63.0)
            res_v[pl.ds(kc * 16, 16)] = dy * dy + dx * dx

    @pl.when(jnp.logical_not(dense_all))
    def _():
        for kc in range(_K // 16):
            res_v[pl.ds(kc * 16, 16)] = jnp.full((16,), _BIG, jnp.float32)

        def chunk_body(c, carry):
            base = c * (2 * _ROW)
            y0f = (2 * c).astype(jnp.float32)
            cabs = jnp.abs(hm_v[pl.ds(base, 16)])
            for j in range(1, 8):
                off = base + (j // 4) * _ROW + (j % 4) * 16
                cabs = jnp.minimum(cabs, jnp.abs(hm_v[pl.ds(off, 16)]))
            dense_c = _all_nonzero(cabs)

            @pl.when(dense_c)
            def _():
                def fast_kc(kc, cc):
                    ky, kx = load_kp(kc)
                    ty = jnp.minimum(jnp.maximum(ky, y0f), y0f + 1.0)
                    ys = jnp.where(ty >= y0f + 0.5, y0f + 1.0, y0f)
                    dy = ky - ys
                    dx = kx - _round_clamp(kx, 0.0, 63.0)
                    d2 = dy * dy + dx * dx
                    res_v[pl.ds(kc * 16, 16)] = jnp.minimum(
                        res_v[pl.ds(kc * 16, 16)], d2)
                    return cc

                lax.fori_loop(0, _K // 16, fast_kc, 0)

            @pl.when(jnp.logical_not(dense_c))
            def _():
                def slow_kc(kc, cc):
                    ky, kx = load_kp(kc)

                    def grp(gi, acc):
                        off = base + (gi // 4) * _ROW + (gi % 4) * 16
                        hv = hm_v[pl.ds(off, 16)]
                        pen_v = jnp.where(hv != 0.0, 0.0, _BIG)
                        ypg = y0f + (gi // 4).astype(jnp.float32)
                        xpg = ((gi % 4) * 16).astype(jnp.float32)
                        for lane in range(16):
                            dy = ky - ypg
                            dx = kx - (xpg + float(lane))
                            acc = jnp.minimum(acc, dy * dy + dx * dx + pen_v[lane])
                        return acc

                    acc = lax.fori_loop(0, 8, grp, jnp.full((16,), _BIG, jnp.float32))
                    res_v[pl.ds(kc * 16, 16)] = jnp.minimum(
                        res_v[pl.ds(kc * 16, 16)], acc)
                    return cc

                lax.fori_loop(0, _K // 16, slow_kc, 0)

            return carry

        lax.fori_loop(0, _CPC, chunk_body, 0)

    pltpu.sync_copy(res_v, out_hbm.at[ch])


def _sc_body(hm_hbm, kp_hbm, out_hbm, hm_v, hm_v2, kp_v, res_v, sem_kp, sem1, sem2):
    wid = lax.axis_index("s") * _NC + lax.axis_index("c")
    second = wid < _BN - _NW
    h_kp = pltpu.async_copy(kp_hbm, kp_v, sem_kp)
    h1 = pltpu.async_copy(hm_hbm.at[pl.ds(wid * _CH, _CH)], hm_v, sem1)

    @pl.when(second)
    def _():
        pltpu.async_copy(hm_hbm.at[pl.ds((_NW + wid) * _CH, _CH)], hm_v2, sem2)

    h_kp.wait()
    h1.wait()
    _do_channel(wid, hm_v, kp_v, res_v, out_hbm)

    @pl.when(second)
    def _():
        pltpu.make_async_copy(
            hm_hbm.at[pl.ds((_NW + wid) * _CH, _CH)], hm_v2, sem2).wait()
        _do_channel(_NW + wid, hm_v2, kp_v, res_v, out_hbm)


def _loss_body(d2_ref, scores_ref, out_ref):
    d = jnp.sqrt(d2_ref[...])
    s = scores_ref[...]
    pos = d < 1.0
    safe_d = jnp.where(pos, d, 0.0)
    safe_s = jnp.where(pos, s, 1.0)
    pos_loss = jnp.sum(
        jnp.where(pos, 10000.0 / (1.0 + jnp.exp(safe_d)) * jnp.log(safe_s), 0.0))
    safe_ns = jnp.where(pos, 0.5, 1.0 - s)
    neg_loss = jnp.sum(jnp.where(pos, 0.0, jnp.log(safe_ns)))
    neg_count = jnp.sum(jnp.logical_not(pos).astype(jnp.float32))
    loss = -pos_loss
    loss = jnp.where(neg_count > 0, loss - 10000.0 / neg_count * neg_loss, loss)
    out_ref[0, 0] = loss


def kernel(all_scores, gt_heatmap, keypoints_list):
    hm_pad = jnp.pad(gt_heatmap, ((0, 0), (0, 0), (0, 0), (0, _ROW - _H)))
    hm_flat = hm_pad.reshape(_BN * _CH)
    kp_flat = keypoints_list.transpose(0, 2, 1).reshape(_B * 2 * _K)
    scores_t = all_scores.transpose(0, 2, 1).reshape(_BN, _K)
    mesh = plsc.VectorSubcoreMesh(
        core_axis_name="c", subcore_axis_name="s", num_cores=_NC, num_subcores=_NS)
    d2 = pl.kernel(
        _sc_body,
        out_type=jax.ShapeDtypeStruct((_BN, _K), jnp.float32),
        mesh=mesh,
        scratch_types=[
            pltpu.VMEM((_CH,), jnp.float32),
            pltpu.VMEM((_CH,), jnp.float32),
            pltpu.VMEM((_B * 2 * _K,), jnp.float32),
            pltpu.VMEM((_K,), jnp.float32),
            pltpu.SemaphoreType.DMA,
            pltpu.SemaphoreType.DMA,
            pltpu.SemaphoreType.DMA,
        ],
    )(hm_flat, kp_flat)
    out = pl.pallas_call(
        _loss_body,
        out_shape=jax.ShapeDtypeStruct((1, 1), jnp.float32),
        in_specs=[
            pl.BlockSpec(memory_space=pltpu.VMEM),
            pl.BlockSpec(memory_space=pltpu.VMEM),
        ],
        out_specs=pl.BlockSpec(memory_space=pltpu.SMEM),
    )(d2, scores_t)
    return out[0, 0]
